# Initial kernel scaffold; baseline (speedup 1.0000x reference)
import jax
import jax.numpy as jnp
from jax.experimental import pallas as pl


def kernel(x, pos, batch, W1, b1):
    raise NotImplementedError("write your pallas kernel here")


# speedup vs baseline: 3.5738x; 3.5738x over previous
import jax
import jax.numpy as jnp
from jax import lax
from jax.experimental import pallas as pl
from jax.experimental.pallas import tpu as pltpu
from jax.experimental.pallas import tpu_sc as plsc

_N = 16384
_D = 128
_RATIO = 4
_M = _N // _RATIO
_K = 16

_BN = 2048
_BQA = _BN // _RATIO
_BQ = 128
_NEG = -3.0e38

_NC = 2
_NS = 16
_NW = _NC * _NS
_QW = _M // _NW
_CQ = 8


def _feat_body(x_ref, pos_ref, qpos_ref, w1a_ref, w1b_ref, b1_ref,
               ya_ref, qc_ref):
    xa = jnp.dot(x_ref[...], w1a_ref[...], preferred_element_type=jnp.float32)
    w = w1b_ref[...]
    p = pos_ref[...]
    pb = p[:, 0:1] * w[0:1, :] + p[:, 1:2] * w[1:2, :] + p[:, 2:3] * w[2:3, :]
    ya_ref[...] = xa + pb
    q = qpos_ref[...]
    qb = q[:, 0:1] * w[0:1, :] + q[:, 1:2] * w[1:2, :] + q[:, 2:3] * w[2:3, :]
    qc_ref[...] = b1_ref[...] - qb


def _topk_body(qpos_ref, post_ref, nbr_ref):
    px = post_ref[0:1, :]
    py = post_ref[1:2, :]
    pz = post_ref[2:3, :]
    p2 = px * px + py * py + pz * pz
    bf = jnp.bfloat16
    pxb = px.astype(bf).astype(jnp.float32)
    pyb = py.astype(bf).astype(jnp.float32)
    pzb = pz.astype(bf).astype(jnp.float32)
    q = qpos_ref[...].astype(bf).astype(jnp.float32)
    d = 2.0 * (q[:, 0:1] * pxb + q[:, 1:2] * pyb + q[:, 2:3] * pzb) - p2
    iota = lax.broadcasted_iota(jnp.int32, (1, _N), 1)
    cols = []
    for _ in range(_K):
        m = jnp.max(d, axis=1, keepdims=True)
        sel = jnp.where(d == m, iota, _N)
        idx = jnp.min(sel, axis=1, keepdims=True)
        cols.append(idx)
        d = jnp.where(iota == idx, _NEG, d)
    nbr_ref[...] = jnp.concatenate(cols, axis=1)


def _gather_max_body(ya_hbm, nbr_hbm, qc_hbm, out_hbm,
                     idx_v, rows_v, qc_v, out_v, sem):
    wid = lax.axis_index("s") * _NC + lax.axis_index("c")
    base = wid * _QW

    def chunk(i, carry):
        q0 = base + i * _CQ
        pltpu.sync_copy(nbr_hbm.at[pl.ds(q0 * _K, _CQ * _K)], idx_v)
        pltpu.async_copy(ya_hbm.at[idx_v], rows_v, sem).wait()
        pltpu.sync_copy(qc_hbm.at[pl.ds(q0, _CQ)], qc_v)
        for q in range(_CQ):
            for g in range(_D // 16):
                sl = pl.ds(g * 16, 16)
                acc = rows_v[q * _K, sl]
                for r in range(1, _K):
                    acc = jnp.maximum(acc, rows_v[q * _K + r, sl])
                acc = acc + qc_v[q, sl]
                out_v[q, sl] = jnp.maximum(acc, 0.0)
        pltpu.sync_copy(out_v, out_hbm.at[pl.ds(q0, _CQ)])
        return carry

    lax.fori_loop(0, _QW // _CQ, chunk, 0)


def kernel(x, pos, batch, W1, b1):
    w1a = W1[:_D]
    w1b = W1[_D:]
    b1r = b1.reshape(1, _D)
    qpos = pos[::_RATIO]
    post = pos.T

    ya, qc = pl.pallas_call(
        _feat_body,
        grid=(_N // _BN,),
        in_specs=[
            pl.BlockSpec((_BN, _D), lambda i: (i, 0)),
            pl.BlockSpec((_BN, 3), lambda i: (i, 0)),
            pl.BlockSpec((_BQA, 3), lambda i: (i, 0)),
            pl.BlockSpec((_D, _D), lambda i: (0, 0)),
            pl.BlockSpec((3, _D), lambda i: (0, 0)),
            pl.BlockSpec((1, _D), lambda i: (0, 0)),
        ],
        out_specs=[
            pl.BlockSpec((_BN, _D), lambda i: (i, 0)),
            pl.BlockSpec((_BQA, _D), lambda i: (i, 0)),
        ],
        out_shape=[
            jax.ShapeDtypeStruct((_N, _D), jnp.float32),
            jax.ShapeDtypeStruct((_M, _D), jnp.float32),
        ],
    )(x, pos, qpos, w1a, w1b, b1r)

    nbr = pl.pallas_call(
        _topk_body,
        grid=(_M // _BQ,),
        in_specs=[
            pl.BlockSpec((_BQ, 3), lambda i: (i, 0)),
            pl.BlockSpec((3, _N), lambda i: (0, 0)),
        ],
        out_specs=pl.BlockSpec((_BQ, _K), lambda i: (i, 0)),
        out_shape=jax.ShapeDtypeStruct((_M, _K), jnp.int32),
    )(qpos, post)

    mesh = plsc.VectorSubcoreMesh(
        core_axis_name="c", subcore_axis_name="s",
        num_cores=_NC, num_subcores=_NS)
    out_x = pl.kernel(
        _gather_max_body,
        out_type=jax.ShapeDtypeStruct((_M, _D), jnp.float32),
        mesh=mesh,
        scratch_types=[
            pltpu.VMEM((_CQ * _K,), jnp.int32),
            pltpu.VMEM((_CQ * _K, _D), jnp.float32),
            pltpu.VMEM((_CQ, _D), jnp.float32),
            pltpu.VMEM((_CQ, _D), jnp.float32),
            pltpu.SemaphoreType.DMA,
        ],
    )(ya, nbr.reshape(_M * _K), qc)

    return out_x, qpos, batch[::_RATIO]


# speedup vs baseline: 13.2425x; 3.7054x over previous
import jax
import jax.numpy as jnp
from jax import lax
from jax.experimental import pallas as pl
from jax.experimental.pallas import tpu as pltpu
from jax.experimental.pallas import tpu_sc as plsc

_N = 16384
_D = 128
_RATIO = 4
_M = _N // _RATIO
_K = 16
_G = _N // _K
_C = _K * _K

_BN = 2048
_BQA = _BN // _RATIO
_BQ = 256
_NEG = -3.0e38

_NC = 2
_NS = 16
_NW = _NC * _NS
_QW = _M // _NW
_CQ = 8


def _feat_body(x_ref, pos_ref, qpos_ref, w1a_ref, w1b_ref, b1_ref,
               ya_ref, qc_ref, ptab_ref, qsplat_ref):
    xa = jnp.dot(x_ref[...], w1a_ref[...], preferred_element_type=jnp.float32)
    w = w1b_ref[...]
    p = pos_ref[...]
    pb = p[:, 0:1] * w[0:1, :] + p[:, 1:2] * w[1:2, :] + p[:, 2:3] * w[2:3, :]
    ya_ref[...] = xa + pb
    q = qpos_ref[...]
    qb = q[:, 0:1] * w[0:1, :] + q[:, 1:2] * w[1:2, :] + q[:, 2:3] * w[2:3, :]
    qc_ref[...] = b1_ref[...] - qb
    bf = jnp.bfloat16
    pbr = p.astype(bf).astype(jnp.float32)
    p2 = (p[:, 0:1] * p[:, 0:1] + p[:, 1:2] * p[:, 1:2]
          + p[:, 2:3] * p[:, 2:3])
    ptab_ref[...] = jnp.concatenate([pbr, p2], axis=1)
    qbr = q.astype(bf).astype(jnp.float32)
    qsplat_ref[...] = jnp.broadcast_to(
        qbr[:, :, None], (q.shape[0], 3, _K)).reshape(q.shape[0], 3 * _K)


def _groupsel_body(qpos_ref, pxpl_ref, pypl_ref, pzpl_ref, p2pl_ref,
                   grp_ref):
    bf = jnp.bfloat16
    q = qpos_ref[...].astype(bf).astype(jnp.float32)
    qx = q[:, 0:1]
    qy = q[:, 1:2]
    qz = q[:, 2:3]
    gmax = None
    for t in range(_K):
        px = pxpl_ref[t:t + 1, :]
        py = pypl_ref[t:t + 1, :]
        pz = pzpl_ref[t:t + 1, :]
        p2 = p2pl_ref[t:t + 1, :]
        dt = 2.0 * (qx * px + qy * py + qz * pz) - p2
        gmax = dt if gmax is None else jnp.maximum(gmax, dt)
    iota = lax.broadcasted_iota(jnp.int32, (1, _G), 1)
    cols = []
    for _ in range(_K):
        m = jnp.max(gmax, axis=1, keepdims=True)
        sel = jnp.where(gmax == m, iota, _G)
        idx = jnp.min(sel, axis=1, keepdims=True)
        cols.append(idx)
        gmax = jnp.where(iota == idx, _NEG, gmax)
    grp_ref[...] = jnp.concatenate(cols, axis=1)


def _cand_body(ptc_hbm, grp_hbm, qs_hbm, dc_hbm,
               gidx_v, ch_v, qs_v, dc_v, sem):
    wid = lax.axis_index("s") * _NC + lax.axis_index("c")
    base = wid * _QW

    def chunk(i, carry):
        q0 = base + i * _CQ
        pltpu.sync_copy(grp_hbm.at[pl.ds(q0 * _K, _CQ * _K)], gidx_v)
        pltpu.async_copy(ptc_hbm.at[gidx_v], ch_v, sem).wait()
        pltpu.sync_copy(qs_hbm.at[pl.ds(q0, _CQ)], qs_v)
        for q in range(_CQ):
            qx = qs_v[q, pl.ds(0, 16)]
            qy = qs_v[q, pl.ds(16, 16)]
            qz = qs_v[q, pl.ds(32, 16)]
            for k in range(_K):
                r = q * _K + k
                xs = ch_v[r, pl.ds(0, 16)]
                ys = ch_v[r, pl.ds(16, 16)]
                zs = ch_v[r, pl.ds(32, 16)]
                p2 = ch_v[r, pl.ds(48, 16)]
                d = 2.0 * (qx * xs + qy * ys + qz * zs) - p2
                dc_v[q, pl.ds(k * 16, 16)] = d
        pltpu.sync_copy(dc_v, dc_hbm.at[pl.ds(q0, _CQ)])
        return carry

    lax.fori_loop(0, _QW // _CQ, chunk, 0)


def _pick_body(grp_ref, dc_ref, nbr_ref):
    g16 = grp_ref[...] * _K
    idxc = (jnp.broadcast_to(g16[:, :, None], (g16.shape[0], _K, _K))
            .reshape(g16.shape[0], _C)
            + lax.broadcasted_iota(jnp.int32, (1, _C), 1) % _K)
    d = dc_ref[...]
    cols = []
    for _ in range(_K):
        m = jnp.max(d, axis=1, keepdims=True)
        sel = jnp.where(d == m, idxc, _N)
        idx = jnp.min(sel, axis=1, keepdims=True)
        cols.append(idx)
        d = jnp.where(idxc == idx, _NEG, d)
    nbr_ref[...] = jnp.concatenate(cols, axis=1)


def _gather_max_body(ya_hbm, nbr_hbm, qc_hbm, out_hbm,
                     idx_v, rows_v, qc_v, out_v, sem):
    wid = lax.axis_index("s") * _NC + lax.axis_index("c")
    base = wid * _QW

    def chunk(i, carry):
        q0 = base + i * _CQ
        pltpu.sync_copy(nbr_hbm.at[pl.ds(q0 * _K, _CQ * _K)], idx_v)
        pltpu.async_copy(ya_hbm.at[idx_v], rows_v, sem).wait()
        pltpu.sync_copy(qc_hbm.at[pl.ds(q0, _CQ)], qc_v)
        for q in range(_CQ):
            for g in range(_D // 16):
                sl = pl.ds(g * 16, 16)
                acc = rows_v[q * _K, sl]
                for r in range(1, _K):
                    acc = jnp.maximum(acc, rows_v[q * _K + r, sl])
                acc = acc + qc_v[q, sl]
                out_v[q, sl] = jnp.maximum(acc, 0.0)
        pltpu.sync_copy(out_v, out_hbm.at[pl.ds(q0, _CQ)])
        return carry

    lax.fori_loop(0, _QW // _CQ, chunk, 0)


def kernel(x, pos, batch, W1, b1):
    w1a = W1[:_D]
    w1b = W1[_D:]
    b1r = b1.reshape(1, _D)
    qpos = pos[::_RATIO]

    ya, qc, ptab, qsplat = pl.pallas_call(
        _feat_body,
        grid=(_N // _BN,),
        in_specs=[
            pl.BlockSpec((_BN, _D), lambda i: (i, 0)),
            pl.BlockSpec((_BN, 3), lambda i: (i, 0)),
            pl.BlockSpec((_BQA, 3), lambda i: (i, 0)),
            pl.BlockSpec((_D, _D), lambda i: (0, 0)),
            pl.BlockSpec((3, _D), lambda i: (0, 0)),
            pl.BlockSpec((1, _D), lambda i: (0, 0)),
        ],
        out_specs=[
            pl.BlockSpec((_BN, _D), lambda i: (i, 0)),
            pl.BlockSpec((_BQA, _D), lambda i: (i, 0)),
            pl.BlockSpec((_BN, 4), lambda i: (i, 0)),
            pl.BlockSpec((_BQA, 3 * _K), lambda i: (i, 0)),
        ],
        out_shape=[
            jax.ShapeDtypeStruct((_N, _D), jnp.float32),
            jax.ShapeDtypeStruct((_M, _D), jnp.float32),
            jax.ShapeDtypeStruct((_N, 4), jnp.float32),
            jax.ShapeDtypeStruct((_M, 3 * _K), jnp.float32),
        ],
    )(x, pos, qpos, w1a, w1b, b1r)

    ptg = ptab.reshape(_G, _K, 4)
    planes = ptg.transpose(2, 1, 0)
    pxpl, pypl, pzpl, p2pl = planes[0], planes[1], planes[2], planes[3]
    ptab_chunks = jnp.pad(ptg.transpose(0, 2, 1).reshape(_G, 4 * _K),
                          ((0, 0), (0, _D - 4 * _K)))

    grp = pl.pallas_call(
        _groupsel_body,
        grid=(_M // _BQ,),
        in_specs=[
            pl.BlockSpec((_BQ, 3), lambda i: (i, 0)),
            pl.BlockSpec((_K, _G), lambda i: (0, 0)),
            pl.BlockSpec((_K, _G), lambda i: (0, 0)),
            pl.BlockSpec((_K, _G), lambda i: (0, 0)),
            pl.BlockSpec((_K, _G), lambda i: (0, 0)),
        ],
        out_specs=pl.BlockSpec((_BQ, _K), lambda i: (i, 0)),
        out_shape=jax.ShapeDtypeStruct((_M, _K), jnp.int32),
    )(qpos, pxpl, pypl, pzpl, p2pl)

    mesh = plsc.VectorSubcoreMesh(
        core_axis_name="c", subcore_axis_name="s",
        num_cores=_NC, num_subcores=_NS)

    dcand = pl.kernel(
        _cand_body,
        out_type=jax.ShapeDtypeStruct((_M, _C), jnp.float32),
        mesh=mesh,
        scratch_types=[
            pltpu.VMEM((_CQ * _K,), jnp.int32),
            pltpu.VMEM((_CQ * _K, _D), jnp.float32),
            pltpu.VMEM((_CQ, 3 * _K), jnp.float32),
            pltpu.VMEM((_CQ, _C), jnp.float32),
            pltpu.SemaphoreType.DMA,
        ],
    )(ptab_chunks, grp.reshape(_M * _K), qsplat)

    nbr = pl.pallas_call(
        _pick_body,
        grid=(_M // _BQ,),
        in_specs=[
            pl.BlockSpec((_BQ, _K), lambda i: (i, 0)),
            pl.BlockSpec((_BQ, _C), lambda i: (i, 0)),
        ],
        out_specs=pl.BlockSpec((_BQ, _K), lambda i: (i, 0)),
        out_shape=jax.ShapeDtypeStruct((_M, _K), jnp.int32),
    )(grp, dcand)

    out_x = pl.kernel(
        _gather_max_body,
        out_type=jax.ShapeDtypeStruct((_M, _D), jnp.float32),
        mesh=mesh,
        scratch_types=[
            pltpu.VMEM((_CQ * _K,), jnp.int32),
            pltpu.VMEM((_CQ * _K, _D), jnp.float32),
            pltpu.VMEM((_CQ, _D), jnp.float32),
            pltpu.VMEM((_CQ, _D), jnp.float32),
            pltpu.SemaphoreType.DMA,
        ],
    )(ya, nbr.reshape(_M * _K), qc)

    return out_x, qpos, batch[::_RATIO]


# speedup vs baseline: 13.9179x; 1.0510x over previous
import jax
import jax.numpy as jnp
from jax import lax
from jax.experimental import pallas as pl
from jax.experimental.pallas import tpu as pltpu
from jax.experimental.pallas import tpu_sc as plsc

_N = 16384
_D = 128
_RATIO = 4
_M = _N // _RATIO
_K = 16
_G = _N // _K
_C = _K * _K

_BN = 2048
_BQA = _BN // _RATIO
_BQ = 256
_NEG = -3.0e38

_NC = 2
_NS = 16
_NW = _NC * _NS
_QW = _M // _NW
_CQ = 8


def _feat_body(x_ref, pos_ref, qpos_ref, w1a_ref, w1b_ref, b1_ref,
               ya_ref, qc_ref, ptab_ref, qsplat_ref):
    xa = jnp.dot(x_ref[...], w1a_ref[...], preferred_element_type=jnp.float32)
    w = w1b_ref[...]
    p = pos_ref[...]
    pb = p[:, 0:1] * w[0:1, :] + p[:, 1:2] * w[1:2, :] + p[:, 2:3] * w[2:3, :]
    ya_ref[...] = xa + pb
    q = qpos_ref[...]
    qb = q[:, 0:1] * w[0:1, :] + q[:, 1:2] * w[1:2, :] + q[:, 2:3] * w[2:3, :]
    qc_ref[...] = b1_ref[...] - qb
    bf = jnp.bfloat16
    pbr = p.astype(bf).astype(jnp.float32)
    p2 = (p[:, 0:1] * p[:, 0:1] + p[:, 1:2] * p[:, 1:2]
          + p[:, 2:3] * p[:, 2:3])
    ptab_ref[...] = jnp.concatenate([pbr, p2], axis=1)
    qbr = q.astype(bf).astype(jnp.float32)
    qsplat_ref[...] = jnp.broadcast_to(
        qbr[:, :, None], (q.shape[0], 3, _K)).reshape(q.shape[0], 3 * _K)


def _groupsel_body(qpad_ref, ppermt_ref, p2perm_ref, grp_ref):
    dot = jnp.dot(qpad_ref[...], ppermt_ref[...],
                  preferred_element_type=jnp.float32)
    p2 = p2perm_ref[...]
    gmax = None
    for t in range(_K):
        sl = slice(t * _G, (t + 1) * _G)
        dt = 2.0 * dot[:, sl] - p2[:, sl]
        gmax = dt if gmax is None else jnp.maximum(gmax, dt)
    iota = lax.broadcasted_iota(jnp.int32, (1, _G), 1)
    cols = []
    for _ in range(_K):
        m = jnp.max(gmax, axis=1, keepdims=True)
        sel = jnp.where(gmax == m, iota, _G)
        idx = jnp.min(sel, axis=1, keepdims=True)
        cols.append(idx)
        gmax = jnp.where(sel == idx, _NEG, gmax)
    grp_ref[...] = jnp.concatenate(cols, axis=1)


def _cand_body(ptc_hbm, grp_hbm, qs_hbm, dc_hbm,
               gidx_v, ch_v, qs_v, dc_v, sem):
    wid = lax.axis_index("s") * _NC + lax.axis_index("c")
    base = wid * _QW

    def chunk(i, carry):
        q0 = base + i * _CQ
        pltpu.sync_copy(grp_hbm.at[pl.ds(q0 * _K, _CQ * _K)], gidx_v)
        pltpu.async_copy(ptc_hbm.at[gidx_v], ch_v, sem).wait()
        pltpu.sync_copy(qs_hbm.at[pl.ds(q0, _CQ)], qs_v)
        for q in range(_CQ):
            qx = qs_v[q, pl.ds(0, 16)]
            qy = qs_v[q, pl.ds(16, 16)]
            qz = qs_v[q, pl.ds(32, 16)]
            for k in range(_K):
                r = q * _K + k
                xs = ch_v[r, pl.ds(0, 16)]
                ys = ch_v[r, pl.ds(16, 16)]
                zs = ch_v[r, pl.ds(32, 16)]
                p2 = ch_v[r, pl.ds(48, 16)]
                d = 2.0 * (qx * xs + qy * ys + qz * zs) - p2
                dc_v[q, pl.ds(k * 16, 16)] = d
        pltpu.sync_copy(dc_v, dc_hbm.at[pl.ds(q0, _CQ)])
        return carry

    lax.fori_loop(0, _QW // _CQ, chunk, 0)


def _pick_body(grp_ref, dc_ref, nbr_ref):
    g16 = grp_ref[...] * _K
    idxc = (jnp.broadcast_to(g16[:, :, None], (g16.shape[0], _K, _K))
            .reshape(g16.shape[0], _C)
            + lax.broadcasted_iota(jnp.int32, (1, _C), 1) % _K)
    d = dc_ref[...]
    cols = []
    for _ in range(_K):
        m = jnp.max(d, axis=1, keepdims=True)
        sel = jnp.where(d == m, idxc, _N)
        idx = jnp.min(sel, axis=1, keepdims=True)
        cols.append(idx)
        d = jnp.where(sel == idx, _NEG, d)
    nbr_ref[...] = jnp.concatenate(cols, axis=1)


def _gather_max_body(ya_hbm, nbr_hbm, qc_hbm, out_hbm,
                     idx_v, rows_v, qc_v, out_v, sem):
    wid = lax.axis_index("s") * _NC + lax.axis_index("c")
    base = wid * _QW

    def chunk(i, carry):
        q0 = base + i * _CQ
        pltpu.sync_copy(nbr_hbm.at[pl.ds(q0 * _K, _CQ * _K)], idx_v)
        pltpu.async_copy(ya_hbm.at[idx_v], rows_v, sem).wait()
        pltpu.sync_copy(qc_hbm.at[pl.ds(q0, _CQ)], qc_v)
        for q in range(_CQ):
            for g in range(_D // 16):
                sl = pl.ds(g * 16, 16)
                acc = rows_v[q * _K, sl]
                for r in range(1, _K):
                    acc = jnp.maximum(acc, rows_v[q * _K + r, sl])
                acc = acc + qc_v[q, sl]
                out_v[q, sl] = jnp.maximum(acc, 0.0)
        pltpu.sync_copy(out_v, out_hbm.at[pl.ds(q0, _CQ)])
        return carry

    lax.fori_loop(0, _QW // _CQ, chunk, 0)


def kernel(x, pos, batch, W1, b1):
    w1a = W1[:_D]
    w1b = W1[_D:]
    b1r = b1.reshape(1, _D)
    qpos = pos[::_RATIO]

    ya, qc, ptab, qsplat = pl.pallas_call(
        _feat_body,
        grid=(_N // _BN,),
        in_specs=[
            pl.BlockSpec((_BN, _D), lambda i: (i, 0)),
            pl.BlockSpec((_BN, 3), lambda i: (i, 0)),
            pl.BlockSpec((_BQA, 3), lambda i: (i, 0)),
            pl.BlockSpec((_D, _D), lambda i: (0, 0)),
            pl.BlockSpec((3, _D), lambda i: (0, 0)),
            pl.BlockSpec((1, _D), lambda i: (0, 0)),
        ],
        out_specs=[
            pl.BlockSpec((_BN, _D), lambda i: (i, 0)),
            pl.BlockSpec((_BQA, _D), lambda i: (i, 0)),
            pl.BlockSpec((_BN, 4), lambda i: (i, 0)),
            pl.BlockSpec((_BQA, 3 * _K), lambda i: (i, 0)),
        ],
        out_shape=[
            jax.ShapeDtypeStruct((_N, _D), jnp.float32),
            jax.ShapeDtypeStruct((_M, _D), jnp.float32),
            jax.ShapeDtypeStruct((_N, 4), jnp.float32),
            jax.ShapeDtypeStruct((_M, 3 * _K), jnp.float32),
        ],
    )(x, pos, qpos, w1a, w1b, b1r)

    ptg = ptab.reshape(_G, _K, 4)
    ptab_chunks = jnp.pad(ptg.transpose(0, 2, 1).reshape(_G, 4 * _K),
                          ((0, 0), (0, _D - 4 * _K)))
    qpad = jnp.pad(qpos, ((0, 0), (0, 5)))
    ppermt = (jnp.pad(pos, ((0, 0), (0, 5)))
              .reshape(_G, _K, 8).transpose(1, 0, 2)
              .reshape(_N, 8).T)
    p2perm = ptab[:, 3].reshape(_G, _K).T.reshape(1, _N)

    grp = pl.pallas_call(
        _groupsel_body,
        grid=(_M // _BQ,),
        in_specs=[
            pl.BlockSpec((_BQ, 8), lambda i: (i, 0)),
            pl.BlockSpec((8, _N), lambda i: (0, 0)),
            pl.BlockSpec((1, _N), lambda i: (0, 0)),
        ],
        out_specs=pl.BlockSpec((_BQ, _K), lambda i: (i, 0)),
        out_shape=jax.ShapeDtypeStruct((_M, _K), jnp.int32),
    )(qpad, ppermt, p2perm)

    mesh = plsc.VectorSubcoreMesh(
        core_axis_name="c", subcore_axis_name="s",
        num_cores=_NC, num_subcores=_NS)

    dcand = pl.kernel(
        _cand_body,
        out_type=jax.ShapeDtypeStruct((_M, _C), jnp.float32),
        mesh=mesh,
        scratch_types=[
            pltpu.VMEM((_CQ * _K,), jnp.int32),
            pltpu.VMEM((_CQ * _K, _D), jnp.float32),
            pltpu.VMEM((_CQ, 3 * _K), jnp.float32),
            pltpu.VMEM((_CQ, _C), jnp.float32),
            pltpu.SemaphoreType.DMA,
        ],
    )(ptab_chunks, grp.reshape(_M * _K), qsplat)

    nbr = pl.pallas_call(
        _pick_body,
        grid=(_M // _BQ,),
        in_specs=[
            pl.BlockSpec((_BQ, _K), lambda i: (i, 0)),
            pl.BlockSpec((_BQ, _C), lambda i: (i, 0)),
        ],
        out_specs=pl.BlockSpec((_BQ, _K), lambda i: (i, 0)),
        out_shape=jax.ShapeDtypeStruct((_M, _K), jnp.int32),
    )(grp, dcand)

    out_x = pl.kernel(
        _gather_max_body,
        out_type=jax.ShapeDtypeStruct((_M, _D), jnp.float32),
        mesh=mesh,
        scratch_types=[
            pltpu.VMEM((_CQ * _K,), jnp.int32),
            pltpu.VMEM((_CQ * _K, _D), jnp.float32),
            pltpu.VMEM((_CQ, _D), jnp.float32),
            pltpu.VMEM((_CQ, _D), jnp.float32),
            pltpu.SemaphoreType.DMA,
        ],
    )(ya, nbr.reshape(_M * _K), qc)

    return out_x, qpos, batch[::_RATIO]


# speedup vs baseline: 15.5342x; 1.1161x over previous
import jax
import jax.numpy as jnp
from jax import lax
from jax.experimental import pallas as pl
from jax.experimental.pallas import tpu as pltpu
from jax.experimental.pallas import tpu_sc as plsc

_N = 16384
_D = 128
_RATIO = 4
_M = _N // _RATIO
_K = 16
_G = _N // _K
_C = _K * _K

_BN = 2048
_BQA = _BN // _RATIO
_BQ = 256
_NEG = -3.0e38

_NC = 2
_NS = 16
_NW = _NC * _NS
_QW = _M // _NW
_CQ = 8


def _feat_body(x_ref, pos_ref, qpos_ref, w1a_ref, w1b_ref, b1_ref,
               ya_ref, qc_ref, ptab_ref, qsplat_ref):
    xa = jnp.dot(x_ref[...], w1a_ref[...], preferred_element_type=jnp.float32)
    w = w1b_ref[...]
    p = pos_ref[...]
    pb = p[:, 0:1] * w[0:1, :] + p[:, 1:2] * w[1:2, :] + p[:, 2:3] * w[2:3, :]
    ya_ref[...] = xa + pb
    q = qpos_ref[...]
    qb = q[:, 0:1] * w[0:1, :] + q[:, 1:2] * w[1:2, :] + q[:, 2:3] * w[2:3, :]
    qc_ref[...] = b1_ref[...] - qb
    bf = jnp.bfloat16
    pbr = p.astype(bf).astype(jnp.float32)
    p2 = (p[:, 0:1] * p[:, 0:1] + p[:, 1:2] * p[:, 1:2]
          + p[:, 2:3] * p[:, 2:3])
    ptab_ref[...] = jnp.concatenate([pbr, p2], axis=1)
    qbr = q.astype(bf).astype(jnp.float32)
    qsplat_ref[...] = jnp.broadcast_to(
        qbr[:, :, None], (q.shape[0], 3, _K)).reshape(q.shape[0], 3 * _K)


def _groupsel_body(qpad_ref, ppermt_ref, p2perm_ref, grp_ref):
    dot = jnp.dot(qpad_ref[...], ppermt_ref[...],
                  preferred_element_type=jnp.float32)
    p2 = p2perm_ref[...]
    gmax = None
    for t in range(_K):
        sl = slice(t * _G, (t + 1) * _G)
        dt = 2.0 * dot[:, sl] - p2[:, sl]
        gmax = dt if gmax is None else jnp.maximum(gmax, dt)
    iota = lax.broadcasted_iota(jnp.int32, (1, _G), 1)
    cols = []
    for _ in range(_K):
        m = jnp.max(gmax, axis=1, keepdims=True)
        sel = jnp.where(gmax == m, iota, _G)
        idx = jnp.min(sel, axis=1, keepdims=True)
        cols.append(idx)
        gmax = jnp.where(sel == idx, _NEG, gmax)
    grp_ref[...] = jnp.concatenate(cols, axis=1)


def _cand_body(ptc_hbm, grp_hbm, qs_hbm, dc_hbm,
               gidx_v, ch0, ch1, qs_v, dc_v, sem0, sem1):
    wid = lax.axis_index("s") * _NC + lax.axis_index("c")
    base = wid * _QW
    nidx = _CQ * _K
    pltpu.sync_copy(grp_hbm.at[pl.ds(base * _K, _QW * _K)], gidx_v)
    pltpu.sync_copy(qs_hbm.at[pl.ds(base, _QW)], qs_v)
    pltpu.async_copy(ptc_hbm.at[gidx_v.at[pl.ds(0, nidx)]], ch0, sem0)

    def compute(ch_v, c):
        for q in range(_CQ):
            row = c * _CQ + q
            qx = qs_v[row, pl.ds(0, 16)]
            qy = qs_v[row, pl.ds(16, 16)]
            qz = qs_v[row, pl.ds(32, 16)]
            for k in range(_K):
                r = q * _K + k
                xs = ch_v[r, pl.ds(0, 16)]
                ys = ch_v[r, pl.ds(16, 16)]
                zs = ch_v[r, pl.ds(32, 16)]
                p2 = ch_v[r, pl.ds(48, 16)]
                d = 2.0 * (qx * xs + qy * ys + qz * zs) - p2
                dc_v[row, pl.ds(k * 16, 16)] = d

    def pair(i, carry):
        c0 = 2 * i
        pltpu.async_copy(
            ptc_hbm.at[gidx_v.at[pl.ds((c0 + 1) * nidx, nidx)]], ch1, sem1)
        pltpu.make_async_copy(
            ptc_hbm.at[gidx_v.at[pl.ds(0, nidx)]], ch0, sem0).wait()
        compute(ch0, c0)

        @pl.when(i < _QW // _CQ // 2 - 1)
        def _():
            pltpu.async_copy(
                ptc_hbm.at[gidx_v.at[pl.ds((c0 + 2) * nidx, nidx)]],
                ch0, sem0)

        pltpu.make_async_copy(
            ptc_hbm.at[gidx_v.at[pl.ds(0, nidx)]], ch1, sem1).wait()
        compute(ch1, c0 + 1)
        return carry

    lax.fori_loop(0, _QW // _CQ // 2, pair, 0)
    pltpu.sync_copy(dc_v, dc_hbm.at[pl.ds(base, _QW)])


def _pick_body(grp_ref, dc_ref, nbr_ref):
    g16 = grp_ref[...] * _K
    idxc = (jnp.broadcast_to(g16[:, :, None], (g16.shape[0], _K, _K))
            .reshape(g16.shape[0], _C)
            + lax.broadcasted_iota(jnp.int32, (1, _C), 1) % _K)
    d = dc_ref[...]
    cols = []
    for _ in range(_K):
        m = jnp.max(d, axis=1, keepdims=True)
        sel = jnp.where(d == m, idxc, _N)
        idx = jnp.min(sel, axis=1, keepdims=True)
        cols.append(idx)
        d = jnp.where(sel == idx, _NEG, d)
    nbr_ref[...] = jnp.concatenate(cols, axis=1)


def _gather_max_body(ya_hbm, nbr_hbm, qc_hbm, out_hbm,
                     idx_v, r0, r1, qc_v, out_v, sem0, sem1):
    wid = lax.axis_index("s") * _NC + lax.axis_index("c")
    base = wid * _QW
    nidx = _CQ * _K
    pltpu.sync_copy(nbr_hbm.at[pl.ds(base * _K, _QW * _K)], idx_v)
    pltpu.sync_copy(qc_hbm.at[pl.ds(base, _QW)], qc_v)
    pltpu.async_copy(ya_hbm.at[idx_v.at[pl.ds(0, nidx)]], r0, sem0)

    def compute(rows_v, c):
        for q in range(_CQ):
            row = c * _CQ + q
            for g in range(_D // 16):
                sl = pl.ds(g * 16, 16)
                acc = rows_v[q * _K, sl]
                for r in range(1, _K):
                    acc = jnp.maximum(acc, rows_v[q * _K + r, sl])
                acc = acc + qc_v[row, sl]
                out_v[row, sl] = jnp.maximum(acc, 0.0)

    def pair(i, carry):
        c0 = 2 * i
        pltpu.async_copy(
            ya_hbm.at[idx_v.at[pl.ds((c0 + 1) * nidx, nidx)]], r1, sem1)
        pltpu.make_async_copy(
            ya_hbm.at[idx_v.at[pl.ds(0, nidx)]], r0, sem0).wait()
        compute(r0, c0)

        @pl.when(i < _QW // _CQ // 2 - 1)
        def _():
            pltpu.async_copy(
                ya_hbm.at[idx_v.at[pl.ds((c0 + 2) * nidx, nidx)]], r0, sem0)

        pltpu.make_async_copy(
            ya_hbm.at[idx_v.at[pl.ds(0, nidx)]], r1, sem1).wait()
        compute(r1, c0 + 1)
        return carry

    lax.fori_loop(0, _QW // _CQ // 2, pair, 0)
    pltpu.sync_copy(out_v, out_hbm.at[pl.ds(base, _QW)])


def kernel(x, pos, batch, W1, b1):
    w1a = W1[:_D]
    w1b = W1[_D:]
    b1r = b1.reshape(1, _D)
    qpos = pos[::_RATIO]

    ya, qc, ptab, qsplat = pl.pallas_call(
        _feat_body,
        grid=(_N // _BN,),
        in_specs=[
            pl.BlockSpec((_BN, _D), lambda i: (i, 0)),
            pl.BlockSpec((_BN, 3), lambda i: (i, 0)),
            pl.BlockSpec((_BQA, 3), lambda i: (i, 0)),
            pl.BlockSpec((_D, _D), lambda i: (0, 0)),
            pl.BlockSpec((3, _D), lambda i: (0, 0)),
            pl.BlockSpec((1, _D), lambda i: (0, 0)),
        ],
        out_specs=[
            pl.BlockSpec((_BN, _D), lambda i: (i, 0)),
            pl.BlockSpec((_BQA, _D), lambda i: (i, 0)),
            pl.BlockSpec((_BN, 4), lambda i: (i, 0)),
            pl.BlockSpec((_BQA, 3 * _K), lambda i: (i, 0)),
        ],
        out_shape=[
            jax.ShapeDtypeStruct((_N, _D), jnp.float32),
            jax.ShapeDtypeStruct((_M, _D), jnp.float32),
            jax.ShapeDtypeStruct((_N, 4), jnp.float32),
            jax.ShapeDtypeStruct((_M, 3 * _K), jnp.float32),
        ],
    )(x, pos, qpos, w1a, w1b, b1r)

    ptg = ptab.reshape(_G, _K, 4)
    ptab_chunks = jnp.pad(ptg.transpose(0, 2, 1).reshape(_G, 4 * _K),
                          ((0, 0), (0, _D - 4 * _K)))
    qpad = jnp.pad(qpos, ((0, 0), (0, 5)))
    ppermt = (jnp.pad(pos, ((0, 0), (0, 5)))
              .reshape(_G, _K, 8).transpose(1, 0, 2)
              .reshape(_N, 8).T)
    p2perm = ptab[:, 3].reshape(_G, _K).T.reshape(1, _N)

    grp = pl.pallas_call(
        _groupsel_body,
        grid=(_M // _BQ,),
        in_specs=[
            pl.BlockSpec((_BQ, 8), lambda i: (i, 0)),
            pl.BlockSpec((8, _N), lambda i: (0, 0)),
            pl.BlockSpec((1, _N), lambda i: (0, 0)),
        ],
        out_specs=pl.BlockSpec((_BQ, _K), lambda i: (i, 0)),
        out_shape=jax.ShapeDtypeStruct((_M, _K), jnp.int32),
    )(qpad, ppermt, p2perm)

    mesh = plsc.VectorSubcoreMesh(
        core_axis_name="c", subcore_axis_name="s",
        num_cores=_NC, num_subcores=_NS)

    dcand = pl.kernel(
        _cand_body,
        out_type=jax.ShapeDtypeStruct((_M, _C), jnp.float32),
        mesh=mesh,
        scratch_types=[
            pltpu.VMEM((_QW * _K,), jnp.int32),
            pltpu.VMEM((_CQ * _K, _D), jnp.float32),
            pltpu.VMEM((_CQ * _K, _D), jnp.float32),
            pltpu.VMEM((_QW, 3 * _K), jnp.float32),
            pltpu.VMEM((_QW, _C), jnp.float32),
            pltpu.SemaphoreType.DMA,
            pltpu.SemaphoreType.DMA,
        ],
    )(ptab_chunks, grp.reshape(_M * _K), qsplat)

    nbr = pl.pallas_call(
        _pick_body,
        grid=(_M // _BQ,),
        in_specs=[
            pl.BlockSpec((_BQ, _K), lambda i: (i, 0)),
            pl.BlockSpec((_BQ, _C), lambda i: (i, 0)),
        ],
        out_specs=pl.BlockSpec((_BQ, _K), lambda i: (i, 0)),
        out_shape=jax.ShapeDtypeStruct((_M, _K), jnp.int32),
    )(grp, dcand)

    out_x = pl.kernel(
        _gather_max_body,
        out_type=jax.ShapeDtypeStruct((_M, _D), jnp.float32),
        mesh=mesh,
        scratch_types=[
            pltpu.VMEM((_QW * _K,), jnp.int32),
            pltpu.VMEM((_CQ * _K, _D), jnp.float32),
            pltpu.VMEM((_CQ * _K, _D), jnp.float32),
            pltpu.VMEM((_QW, _D), jnp.float32),
            pltpu.VMEM((_QW, _D), jnp.float32),
            pltpu.SemaphoreType.DMA,
            pltpu.SemaphoreType.DMA,
        ],
    )(ya, nbr.reshape(_M * _K), qc)

    return out_x, qpos, batch[::_RATIO]


# speedup vs baseline: 15.6479x; 1.0073x over previous
import jax
import jax.numpy as jnp
from jax import lax
from jax.experimental import pallas as pl
from jax.experimental.pallas import tpu as pltpu
from jax.experimental.pallas import tpu_sc as plsc

_N = 16384
_D = 128
_RATIO = 4
_M = _N // _RATIO
_K = 16
_G = _N // _K
_C = _K * _K

_BN = 2048
_BQA = _BN // _RATIO
_BQ = 256
_NEG = -3.0e38

_NC = 2
_NS = 16
_NW = _NC * _NS
_QW = _M // _NW
_CQ = 8


def _feat_body(x_ref, pos_ref, qpad_ref, w1a_ref, w1b_ref, b1_ref,
               ya_ref, qc_ref, ptab_ref, qsplat_ref):
    xa = jnp.dot(x_ref[...], w1a_ref[...], preferred_element_type=jnp.float32)
    p8 = pos_ref[...]
    pb = jnp.dot(p8, w1b_ref[...], preferred_element_type=jnp.float32)
    ya_ref[...] = xa + pb
    q8 = qpad_ref[...]
    qb = jnp.dot(q8, w1b_ref[...], preferred_element_type=jnp.float32)
    qc_ref[...] = b1_ref[...] - qb
    bf = jnp.bfloat16
    p = p8[:, 0:3]
    pbr = p.astype(bf).astype(jnp.float32)
    p2 = (p[:, 0:1] * p[:, 0:1] + p[:, 1:2] * p[:, 1:2]
          + p[:, 2:3] * p[:, 2:3])
    ptab_ref[...] = jnp.concatenate([pbr, p2], axis=1)
    qbr = q8[:, 0:3].astype(bf).astype(jnp.float32)
    qsplat_ref[...] = jnp.broadcast_to(
        qbr[:, :, None], (q8.shape[0], 3, _K)).reshape(q8.shape[0], 3 * _K)


def _groupsel_body(qpad_ref, ppermt_ref, grp_ref):
    pp = ppermt_ref[...]
    dot = jnp.dot(qpad_ref[...], pp, preferred_element_type=jnp.float32)
    p2 = (pp[0:1, :] * pp[0:1, :] + pp[1:2, :] * pp[1:2, :]
          + pp[2:3, :] * pp[2:3, :])
    gmax = None
    for t in range(_K):
        sl = slice(t * _G, (t + 1) * _G)
        dt = 2.0 * dot[:, sl] - p2[:, sl]
        gmax = dt if gmax is None else jnp.maximum(gmax, dt)
    iota = lax.broadcasted_iota(jnp.int32, (1, _G), 1)
    cols = []
    for _ in range(_K):
        m = jnp.max(gmax, axis=1, keepdims=True)
        sel = jnp.where(gmax == m, iota, _G)
        idx = jnp.min(sel, axis=1, keepdims=True)
        cols.append(idx)
        gmax = jnp.where(sel == idx, _NEG, gmax)
    grp_ref[...] = jnp.concatenate(cols, axis=1)


def _cand_body(ptc_hbm, grp_hbm, qs_hbm, dc_hbm,
               gidx_v, ch0, ch1, qs_v, dc_v, sem0, sem1):
    wid = lax.axis_index("s") * _NC + lax.axis_index("c")
    base = wid * _QW
    nidx = _CQ * _K
    pltpu.sync_copy(grp_hbm.at[pl.ds(base * _K, _QW * _K)], gidx_v)
    pltpu.sync_copy(qs_hbm.at[pl.ds(base, _QW)], qs_v)
    pltpu.async_copy(ptc_hbm.at[gidx_v.at[pl.ds(0, nidx)]], ch0, sem0)

    def compute(ch_v, c):
        for q in range(_CQ):
            row = c * _CQ + q
            qx = qs_v[row, pl.ds(0, 16)]
            qy = qs_v[row, pl.ds(16, 16)]
            qz = qs_v[row, pl.ds(32, 16)]
            for k in range(_K):
                r = q * _K + k
                xs = ch_v[r, pl.ds(0, 16)]
                ys = ch_v[r, pl.ds(16, 16)]
                zs = ch_v[r, pl.ds(32, 16)]
                p2 = ch_v[r, pl.ds(48, 16)]
                d = 2.0 * (qx * xs + qy * ys + qz * zs) - p2
                dc_v[row, pl.ds(k * 16, 16)] = d

    def pair(i, carry):
        c0 = 2 * i
        pltpu.async_copy(
            ptc_hbm.at[gidx_v.at[pl.ds((c0 + 1) * nidx, nidx)]], ch1, sem1)
        pltpu.make_async_copy(
            ptc_hbm.at[gidx_v.at[pl.ds(0, nidx)]], ch0, sem0).wait()
        compute(ch0, c0)

        @pl.when(i < _QW // _CQ // 2 - 1)
        def _():
            pltpu.async_copy(
                ptc_hbm.at[gidx_v.at[pl.ds((c0 + 2) * nidx, nidx)]],
                ch0, sem0)

        pltpu.make_async_copy(
            ptc_hbm.at[gidx_v.at[pl.ds(0, nidx)]], ch1, sem1).wait()
        compute(ch1, c0 + 1)
        return carry

    lax.fori_loop(0, _QW // _CQ // 2, pair, 0)
    pltpu.sync_copy(dc_v, dc_hbm.at[pl.ds(base, _QW)])


def _pick_body(grp_ref, dc_ref, nbr_ref):
    g16 = grp_ref[...] * _K
    idxc = (jnp.broadcast_to(g16[:, :, None], (g16.shape[0], _K, _K))
            .reshape(g16.shape[0], _C)
            + lax.broadcasted_iota(jnp.int32, (1, _C), 1) % _K)
    d = dc_ref[...]
    cols = []
    for _ in range(_K):
        m = jnp.max(d, axis=1, keepdims=True)
        sel = jnp.where(d == m, idxc, _N)
        idx = jnp.min(sel, axis=1, keepdims=True)
        cols.append(idx)
        d = jnp.where(sel == idx, _NEG, d)
    nbr_ref[...] = jnp.concatenate(cols, axis=1)


def _gather_max_body(ya_hbm, nbr_hbm, qc_hbm, out_hbm,
                     idx_v, r0, r1, qc_v, out_v, sem0, sem1):
    wid = lax.axis_index("s") * _NC + lax.axis_index("c")
    base = wid * _QW
    nidx = _CQ * _K
    pltpu.sync_copy(nbr_hbm.at[pl.ds(base * _K, _QW * _K)], idx_v)
    pltpu.sync_copy(qc_hbm.at[pl.ds(base, _QW)], qc_v)
    pltpu.async_copy(ya_hbm.at[idx_v.at[pl.ds(0, nidx)]], r0, sem0)

    def compute(rows_v, c):
        for q in range(_CQ):
            row = c * _CQ + q
            for g in range(_D // 16):
                sl = pl.ds(g * 16, 16)
                acc = rows_v[q * _K, sl]
                for r in range(1, _K):
                    acc = jnp.maximum(acc, rows_v[q * _K + r, sl])
                acc = acc + qc_v[row, sl]
                out_v[row, sl] = jnp.maximum(acc, 0.0)

    def pair(i, carry):
        c0 = 2 * i
        pltpu.async_copy(
            ya_hbm.at[idx_v.at[pl.ds((c0 + 1) * nidx, nidx)]], r1, sem1)
        pltpu.make_async_copy(
            ya_hbm.at[idx_v.at[pl.ds(0, nidx)]], r0, sem0).wait()
        compute(r0, c0)

        @pl.when(i < _QW // _CQ // 2 - 1)
        def _():
            pltpu.async_copy(
                ya_hbm.at[idx_v.at[pl.ds((c0 + 2) * nidx, nidx)]], r0, sem0)

        pltpu.make_async_copy(
            ya_hbm.at[idx_v.at[pl.ds(0, nidx)]], r1, sem1).wait()
        compute(r1, c0 + 1)
        return carry

    lax.fori_loop(0, _QW // _CQ // 2, pair, 0)
    pltpu.sync_copy(out_v, out_hbm.at[pl.ds(base, _QW)])


def kernel(x, pos, batch, W1, b1):
    w1a = W1[:_D]
    w1b_pad = jnp.pad(W1[_D:], ((0, 5), (0, 0)))
    b1r = b1.reshape(1, _D)
    qpos = pos[::_RATIO]
    pos_pad = jnp.pad(pos, ((0, 0), (0, 5)))
    qpad = pos_pad[::_RATIO]

    ya, qc, ptab, qsplat = pl.pallas_call(
        _feat_body,
        grid=(_N // _BN,),
        in_specs=[
            pl.BlockSpec((_BN, _D), lambda i: (i, 0)),
            pl.BlockSpec((_BN, 8), lambda i: (i, 0)),
            pl.BlockSpec((_BQA, 8), lambda i: (i, 0)),
            pl.BlockSpec((_D, _D), lambda i: (0, 0)),
            pl.BlockSpec((8, _D), lambda i: (0, 0)),
            pl.BlockSpec((1, _D), lambda i: (0, 0)),
        ],
        out_specs=[
            pl.BlockSpec((_BN, _D), lambda i: (i, 0)),
            pl.BlockSpec((_BQA, _D), lambda i: (i, 0)),
            pl.BlockSpec((_BN, 4), lambda i: (i, 0)),
            pl.BlockSpec((_BQA, 3 * _K), lambda i: (i, 0)),
        ],
        out_shape=[
            jax.ShapeDtypeStruct((_N, _D), jnp.float32),
            jax.ShapeDtypeStruct((_M, _D), jnp.float32),
            jax.ShapeDtypeStruct((_N, 4), jnp.float32),
            jax.ShapeDtypeStruct((_M, 3 * _K), jnp.float32),
        ],
    )(x, pos_pad, qpad, w1a, w1b_pad, b1r)

    ptg = ptab.reshape(_G, _K, 4)
    ptab_chunks = jnp.pad(ptg.transpose(0, 2, 1).reshape(_G, 4 * _K),
                          ((0, 0), (0, _D - 4 * _K)))
    ppermt = (pos_pad.reshape(_G, _K, 8).transpose(1, 0, 2)
              .reshape(_N, 8).T)

    grp = pl.pallas_call(
        _groupsel_body,
        grid=(_M // _BQ,),
        in_specs=[
            pl.BlockSpec((_BQ, 8), lambda i: (i, 0)),
            pl.BlockSpec((8, _N), lambda i: (0, 0)),
        ],
        out_specs=pl.BlockSpec((_BQ, _K), lambda i: (i, 0)),
        out_shape=jax.ShapeDtypeStruct((_M, _K), jnp.int32),
    )(qpad, ppermt)

    mesh = plsc.VectorSubcoreMesh(
        core_axis_name="c", subcore_axis_name="s",
        num_cores=_NC, num_subcores=_NS)

    dcand = pl.kernel(
        _cand_body,
        out_type=jax.ShapeDtypeStruct((_M, _C), jnp.float32),
        mesh=mesh,
        scratch_types=[
            pltpu.VMEM((_QW * _K,), jnp.int32),
            pltpu.VMEM((_CQ * _K, _D), jnp.float32),
            pltpu.VMEM((_CQ * _K, _D), jnp.float32),
            pltpu.VMEM((_QW, 3 * _K), jnp.float32),
            pltpu.VMEM((_QW, _C), jnp.float32),
            pltpu.SemaphoreType.DMA,
            pltpu.SemaphoreType.DMA,
        ],
    )(ptab_chunks, grp.reshape(_M * _K), qsplat)

    nbr = pl.pallas_call(
        _pick_body,
        grid=(_M // _BQ,),
        in_specs=[
            pl.BlockSpec((_BQ, _K), lambda i: (i, 0)),
            pl.BlockSpec((_BQ, _C), lambda i: (i, 0)),
        ],
        out_specs=pl.BlockSpec((_BQ, _K), lambda i: (i, 0)),
        out_shape=jax.ShapeDtypeStruct((_M, _K), jnp.int32),
    )(grp, dcand)

    out_x = pl.kernel(
        _gather_max_body,
        out_type=jax.ShapeDtypeStruct((_M, _D), jnp.float32),
        mesh=mesh,
        scratch_types=[
            pltpu.VMEM((_QW * _K,), jnp.int32),
            pltpu.VMEM((_CQ * _K, _D), jnp.float32),
            pltpu.VMEM((_CQ * _K, _D), jnp.float32),
            pltpu.VMEM((_QW, _D), jnp.float32),
            pltpu.VMEM((_QW, _D), jnp.float32),
            pltpu.SemaphoreType.DMA,
            pltpu.SemaphoreType.DMA,
        ],
    )(ya, nbr.reshape(_M * _K), qc)

    return out_x, qpos, batch[::_RATIO]


# speedup vs baseline: 20.4165x; 1.3047x over previous
import jax
import jax.numpy as jnp
from jax import lax
from jax.experimental import pallas as pl
from jax.experimental.pallas import tpu as pltpu
from jax.experimental.pallas import tpu_sc as plsc

_N = 16384
_D = 128
_RATIO = 4
_M = _N // _RATIO
_K = 16
_G = _N // _K
_C = _K * _K

_BN = 2048
_BQA = _BN // _RATIO
_BQ = 512
_BQ2 = 1024
_NEG = -3.0e38

_NC = 2
_NS = 16
_NW = _NC * _NS
_QW = _M // _NW
_CQ = 8


def _feat_body(x_ref, pos_ref, qpad_ref, w1a_ref, w1b_ref, b1_ref,
               ya_ref, qc_ref, ptab_ref, qsplat_ref):
    xa = jnp.dot(x_ref[...], w1a_ref[...], preferred_element_type=jnp.float32)
    p8 = pos_ref[...]
    pb = jnp.dot(p8, w1b_ref[...], preferred_element_type=jnp.float32)
    ya_ref[...] = xa + pb
    q8 = qpad_ref[...]
    qb = jnp.dot(q8, w1b_ref[...], preferred_element_type=jnp.float32)
    qc_ref[...] = b1_ref[...] - qb
    bf = jnp.bfloat16
    p = p8[:, 0:3]
    pbr = p.astype(bf).astype(jnp.float32)
    p2 = (p[:, 0:1] * p[:, 0:1] + p[:, 1:2] * p[:, 1:2]
          + p[:, 2:3] * p[:, 2:3])
    ptab_ref[...] = jnp.concatenate([pbr, p2], axis=1)
    qbr = q8[:, 0:3].astype(bf).astype(jnp.float32)
    qsplat_ref[...] = jnp.broadcast_to(
        qbr[:, :, None], (q8.shape[0], 3, _K)).reshape(q8.shape[0], 3 * _K)


def _groupsel_body(qpad_ref, ppermt_ref, grp_ref):
    pp = ppermt_ref[...]
    dot = jnp.dot(qpad_ref[...], pp, preferred_element_type=jnp.float32)
    p2 = (pp[0:1, :] * pp[0:1, :] + pp[1:2, :] * pp[1:2, :]
          + pp[2:3, :] * pp[2:3, :])
    gmax = None
    for t in range(_K):
        sl = slice(t * _G, (t + 1) * _G)
        dt = 2.0 * dot[:, sl] - p2[:, sl]
        gmax = dt if gmax is None else jnp.maximum(gmax, dt)
    iota = lax.broadcasted_iota(jnp.int32, (1, _G), 1)
    cols = []
    for _ in range(_K):
        m = jnp.max(gmax, axis=1, keepdims=True)
        sel = jnp.where(gmax == m, iota, _G)
        idx = jnp.min(sel, axis=1, keepdims=True)
        cols.append(idx)
        gmax = jnp.where(sel == idx, _NEG, gmax)
    grp_ref[...] = jnp.concatenate(cols, axis=1)


def _make_cand_body(qw):
  def _cand_body(ptc_hbm, grp_hbm, qs_hbm, dc_hbm,
               gidx_v, ch0, ch1, qs_v, dc_v, sem0, sem1):
    wid = lax.axis_index("s") * _NC + lax.axis_index("c")
    base = wid * qw
    nidx = _CQ * _K
    pltpu.sync_copy(grp_hbm.at[pl.ds(base * _K, qw * _K)], gidx_v)
    pltpu.sync_copy(qs_hbm.at[pl.ds(base, qw)], qs_v)
    pltpu.async_copy(ptc_hbm.at[gidx_v.at[pl.ds(0, nidx)]], ch0, sem0)

    def compute(ch_v, c):
        for q in range(_CQ):
            row = c * _CQ + q
            qx = qs_v[row, pl.ds(0, 16)]
            qy = qs_v[row, pl.ds(16, 16)]
            qz = qs_v[row, pl.ds(32, 16)]
            for k in range(_K):
                r = q * _K + k
                xs = ch_v[r, pl.ds(0, 16)]
                ys = ch_v[r, pl.ds(16, 16)]
                zs = ch_v[r, pl.ds(32, 16)]
                p2 = ch_v[r, pl.ds(48, 16)]
                d = 2.0 * (qx * xs + qy * ys + qz * zs) - p2
                dc_v[row, pl.ds(k * 16, 16)] = d

    def pair(i, carry):
        c0 = 2 * i
        pltpu.async_copy(
            ptc_hbm.at[gidx_v.at[pl.ds((c0 + 1) * nidx, nidx)]], ch1, sem1)
        pltpu.make_async_copy(
            ptc_hbm.at[gidx_v.at[pl.ds(0, nidx)]], ch0, sem0).wait()
        compute(ch0, c0)

        @pl.when(i < qw // _CQ // 2 - 1)
        def _():
            pltpu.async_copy(
                ptc_hbm.at[gidx_v.at[pl.ds((c0 + 2) * nidx, nidx)]],
                ch0, sem0)

        pltpu.make_async_copy(
            ptc_hbm.at[gidx_v.at[pl.ds(0, nidx)]], ch1, sem1).wait()
        compute(ch1, c0 + 1)
        return carry

    lax.fori_loop(0, qw // _CQ // 2, pair, 0)
    pltpu.sync_copy(dc_v, dc_hbm.at[pl.ds(base, qw)])
  return _cand_body


def _pick_body(grp_ref, dc_ref, nbr_ref):
    g16 = grp_ref[...] * _K
    idxc = (jnp.broadcast_to(g16[:, :, None], (g16.shape[0], _K, _K))
            .reshape(g16.shape[0], _C)
            + lax.broadcasted_iota(jnp.int32, (1, _C), 1) % _K)
    d = dc_ref[...]
    cols = []
    for _ in range(_K):
        m = jnp.max(d, axis=1, keepdims=True)
        sel = jnp.where(d == m, idxc, _N)
        idx = jnp.min(sel, axis=1, keepdims=True)
        cols.append(idx)
        d = jnp.where(sel == idx, _NEG, d)
    nbr_ref[...] = jnp.concatenate(cols, axis=1)


def _make_gather_max_body(qw):
  def _gather_max_body(ya_hbm, nbr_hbm, qc_hbm, out_hbm,
                     idx_v, r0, r1, qc_v, out_v, sem0, sem1):
    wid = lax.axis_index("s") * _NC + lax.axis_index("c")
    base = wid * qw
    nidx = _CQ * _K
    pltpu.sync_copy(nbr_hbm.at[pl.ds(base * _K, qw * _K)], idx_v)
    pltpu.sync_copy(qc_hbm.at[pl.ds(base, qw)], qc_v)
    pltpu.async_copy(ya_hbm.at[idx_v.at[pl.ds(0, nidx)]], r0, sem0)

    def compute(rows_v, c):
        for q in range(_CQ):
            row = c * _CQ + q
            for g in range(_D // 16):
                sl = pl.ds(g * 16, 16)
                acc = rows_v[q * _K, sl]
                for r in range(1, _K):
                    acc = jnp.maximum(acc, rows_v[q * _K + r, sl])
                acc = acc + qc_v[row, sl]
                out_v[row, sl] = jnp.maximum(acc, 0.0)

    def pair(i, carry):
        c0 = 2 * i
        pltpu.async_copy(
            ya_hbm.at[idx_v.at[pl.ds((c0 + 1) * nidx, nidx)]], r1, sem1)
        pltpu.make_async_copy(
            ya_hbm.at[idx_v.at[pl.ds(0, nidx)]], r0, sem0).wait()
        compute(r0, c0)

        @pl.when(i < qw // _CQ // 2 - 1)
        def _():
            pltpu.async_copy(
                ya_hbm.at[idx_v.at[pl.ds((c0 + 2) * nidx, nidx)]], r0, sem0)

        pltpu.make_async_copy(
            ya_hbm.at[idx_v.at[pl.ds(0, nidx)]], r1, sem1).wait()
        compute(r1, c0 + 1)
        return carry

    lax.fori_loop(0, qw // _CQ // 2, pair, 0)
    pltpu.sync_copy(out_v, out_hbm.at[pl.ds(base, qw)])
  return _gather_max_body


def kernel(x, pos, batch, W1, b1):
    w1a = W1[:_D]
    w1b_pad = jnp.pad(W1[_D:], ((0, 5), (0, 0)))
    b1r = b1.reshape(1, _D)
    qpos = pos[::_RATIO]
    pos_pad = jnp.pad(pos, ((0, 0), (0, 5)))
    qpad = pos_pad[::_RATIO]

    ya, qc, ptab, qsplat = pl.pallas_call(
        _feat_body,
        grid=(_N // _BN,),
        in_specs=[
            pl.BlockSpec((_BN, _D), lambda i: (i, 0)),
            pl.BlockSpec((_BN, 8), lambda i: (i, 0)),
            pl.BlockSpec((_BQA, 8), lambda i: (i, 0)),
            pl.BlockSpec((_D, _D), lambda i: (0, 0)),
            pl.BlockSpec((8, _D), lambda i: (0, 0)),
            pl.BlockSpec((1, _D), lambda i: (0, 0)),
        ],
        out_specs=[
            pl.BlockSpec((_BN, _D), lambda i: (i, 0)),
            pl.BlockSpec((_BQA, _D), lambda i: (i, 0)),
            pl.BlockSpec((_BN, 4), lambda i: (i, 0)),
            pl.BlockSpec((_BQA, 3 * _K), lambda i: (i, 0)),
        ],
        out_shape=[
            jax.ShapeDtypeStruct((_N, _D), jnp.float32),
            jax.ShapeDtypeStruct((_M, _D), jnp.float32),
            jax.ShapeDtypeStruct((_N, 4), jnp.float32),
            jax.ShapeDtypeStruct((_M, 3 * _K), jnp.float32),
        ],
    )(x, pos_pad, qpad, w1a, w1b_pad, b1r)

    ptg = ptab.reshape(_G, _K, 4)
    ptab_chunks = jnp.pad(ptg.transpose(0, 2, 1).reshape(_G, 4 * _K),
                          ((0, 0), (0, _D - 4 * _K)))
    ppermt = (pos_pad.reshape(_G, _K, 8).transpose(1, 0, 2)
              .reshape(_N, 8).T)

    mesh = plsc.VectorSubcoreMesh(
        core_axis_name="c", subcore_axis_name="s",
        num_cores=_NC, num_subcores=_NS)

    mh = _M // 2
    qwh = mh // _NW
    outs = []
    for h in range(2):
        qpad_h = qpad[h * mh:(h + 1) * mh]
        grp = pl.pallas_call(
            _groupsel_body,
            grid=(mh // _BQ,),
            in_specs=[
                pl.BlockSpec((_BQ, 8), lambda i: (i, 0)),
                pl.BlockSpec((8, _N), lambda i: (0, 0)),
            ],
            out_specs=pl.BlockSpec((_BQ, _K), lambda i: (i, 0)),
            out_shape=jax.ShapeDtypeStruct((mh, _K), jnp.int32),
        )(qpad_h, ppermt)

        dcand = pl.kernel(
            _make_cand_body(qwh),
            out_type=jax.ShapeDtypeStruct((mh, _C), jnp.float32),
            mesh=mesh,
            scratch_types=[
                pltpu.VMEM((qwh * _K,), jnp.int32),
                pltpu.VMEM((_CQ * _K, _D), jnp.float32),
                pltpu.VMEM((_CQ * _K, _D), jnp.float32),
                pltpu.VMEM((qwh, 3 * _K), jnp.float32),
                pltpu.VMEM((qwh, _C), jnp.float32),
                pltpu.SemaphoreType.DMA,
                pltpu.SemaphoreType.DMA,
            ],
        )(ptab_chunks, grp.reshape(mh * _K), qsplat[h * mh:(h + 1) * mh])

        nbr = pl.pallas_call(
            _pick_body,
            grid=(mh // _BQ2,),
            in_specs=[
                pl.BlockSpec((_BQ2, _K), lambda i: (i, 0)),
                pl.BlockSpec((_BQ2, _C), lambda i: (i, 0)),
            ],
            out_specs=pl.BlockSpec((_BQ2, _K), lambda i: (i, 0)),
            out_shape=jax.ShapeDtypeStruct((mh, _K), jnp.int32),
        )(grp, dcand)

        out_h = pl.kernel(
            _make_gather_max_body(qwh),
            out_type=jax.ShapeDtypeStruct((mh, _D), jnp.float32),
            mesh=mesh,
            scratch_types=[
                pltpu.VMEM((qwh * _K,), jnp.int32),
                pltpu.VMEM((_CQ * _K, _D), jnp.float32),
                pltpu.VMEM((_CQ * _K, _D), jnp.float32),
                pltpu.VMEM((qwh, _D), jnp.float32),
                pltpu.VMEM((qwh, _D), jnp.float32),
                pltpu.SemaphoreType.DMA,
                pltpu.SemaphoreType.DMA,
            ],
        )(ya, nbr.reshape(mh * _K), qc[h * mh:(h + 1) * mh])
        outs.append(out_h)

    out_x = jnp.concatenate(outs, axis=0)
    return out_x, qpos, batch[::_RATIO]
